# baseline (speedup 1.0000x reference)
import jax
import jax.numpy as jnp
from jax.experimental import pallas as pl
from jax.experimental.pallas import tpu as pltpu

_N = 10000
_D = 128
_H = 128
_C = 2
_BM = 400


def _graphcad_kernel(adj_ref, f_ref, gamma_ref, beta_ref, w1_ref, b1_ref,
                     a1_ref, w2_ref, b2_ref, a2_ref, w3_ref, b3_ref,
                     out_ref, xbn_ref, y1_ref):
    phase = pl.program_id(0)
    i = pl.program_id(1)

    @pl.when(jnp.logical_and(phase == 0, i == 0))
    def _():
        f = f_ref[...]
        mu = jnp.mean(f, axis=0, keepdims=True)
        var = jnp.mean((f - mu) * (f - mu), axis=0, keepdims=True)
        scale = gamma_ref[...] * jax.lax.rsqrt(var + 1e-5)
        xbn_ref[...] = (f - mu) * scale + beta_ref[...]

    @pl.when(phase == 0)
    def _():
        y1_ref[pl.ds(i * _BM, _BM), :] = jnp.dot(
            adj_ref[...], xbn_ref[...], preferred_element_type=jnp.float32)

    @pl.when(phase == 1)
    def _():
        y2 = jnp.dot(adj_ref[...], y1_ref[...],
                     preferred_element_type=jnp.float32)
        h = y2 @ w1_ref[...] + b1_ref[...]
        h = jnp.where(h >= 0, h, a1_ref[0, 0] * h)
        h = h @ w2_ref[...] + b2_ref[...]
        h = jnp.where(h >= 0, h, a2_ref[0, 0] * h)
        logits = h @ w3_ref[...] + b3_ref[...]
        m = jnp.max(logits, axis=1, keepdims=True)
        s = jnp.sum(jnp.exp(logits - m), axis=1, keepdims=True)
        out_ref[...] = logits - m - jnp.log(s)


def kernel(feature, adj, gamma, beta, W1, b1, a1, W2, b2, a2, W3, b3):
    gamma2 = gamma.reshape(1, _D)
    beta2 = beta.reshape(1, _D)
    b1r = b1.reshape(1, _H)
    b2r = b2.reshape(1, _H)
    a1r = a1.reshape(1, 1)
    a2r = a2.reshape(1, 1)
    b3r = b3.reshape(1, _C)

    const = lambda p, i: (0, 0)
    out = pl.pallas_call(
        _graphcad_kernel,
        grid=(2, _N // _BM),
        in_specs=[
            pl.BlockSpec((_BM, _N), lambda p, i: (i, 0)),
            pl.BlockSpec((_N, _D), const),
            pl.BlockSpec((1, _D), const),
            pl.BlockSpec((1, _D), const),
            pl.BlockSpec((_D, _H), const),
            pl.BlockSpec((1, _H), const),
            pl.BlockSpec((1, 1), const),
            pl.BlockSpec((_H, _H), const),
            pl.BlockSpec((1, _H), const),
            pl.BlockSpec((1, 1), const),
            pl.BlockSpec((_H, _C), const),
            pl.BlockSpec((1, _C), const),
        ],
        out_specs=pl.BlockSpec((_BM, _C), lambda p, i: (i, 0)),
        out_shape=jax.ShapeDtypeStruct((_N, _C), jnp.float32),
        scratch_shapes=[
            pltpu.VMEM((_N, _D), jnp.float32),
            pltpu.VMEM((_N, _D), jnp.float32),
        ],
    )(adj, feature, gamma2, beta2, W1, b1r, a1r, W2, b2r, a2r, W3, b3r)

    return out


# speedup vs baseline: 1.0069x; 1.0069x over previous
import jax
import jax.numpy as jnp
from jax.experimental import pallas as pl
from jax.experimental.pallas import tpu as pltpu

_N = 10000
_D = 128
_H = 128
_C = 2
_BM = 400


def _graphcad_kernel(adj_ref, f_ref, gamma_ref, beta_ref, w1_ref, b1_ref,
                     a1_ref, w2_ref, b2_ref, a2_ref, w3_ref, b3_ref,
                     out_ref, xbn_ref, y1_ref):
    phase = pl.program_id(0)
    i = pl.program_id(1)

    @pl.when(jnp.logical_and(phase == 0, i == 0))
    def _():
        f = f_ref[...]
        mu = jnp.mean(f, axis=0, keepdims=True)
        var = jnp.mean((f - mu) * (f - mu), axis=0, keepdims=True)
        scale = gamma_ref[...] * jax.lax.rsqrt(var + 1e-5)
        xbn_ref[...] = (f - mu) * scale + beta_ref[...]

    @pl.when(phase == 0)
    def _():
        y1_ref[pl.ds(i * _BM, _BM), :] = jnp.dot(
            adj_ref[...], xbn_ref[...], preferred_element_type=jnp.float32)

    @pl.when(phase == 1)
    def _():
        y2 = jnp.dot(adj_ref[...], y1_ref[...],
                     preferred_element_type=jnp.float32)
        h = y2 @ w1_ref[...] + b1_ref[...]
        h = jnp.where(h >= 0, h, a1_ref[0, 0] * h)
        h = h @ w2_ref[...] + b2_ref[...]
        h = jnp.where(h >= 0, h, a2_ref[0, 0] * h)
        logits = h @ w3_ref[...] + b3_ref[...]
        m = jnp.max(logits, axis=1, keepdims=True)
        s = jnp.sum(jnp.exp(logits - m), axis=1, keepdims=True)
        out_ref[...] = logits - m - jnp.log(s)


def kernel(feature, adj, gamma, beta, W1, b1, a1, W2, b2, a2, W3, b3):
    gamma2 = gamma.reshape(1, _D)
    beta2 = beta.reshape(1, _D)
    b1r = b1.reshape(1, _H)
    b2r = b2.reshape(1, _H)
    a1r = a1.reshape(1, 1)
    a2r = a2.reshape(1, 1)
    b3r = b3.reshape(1, _C)

    const = lambda p, i: (0, 0)
    nblk = _N // _BM
    rev = lambda p, i: (p * (nblk - 1 - 2 * i) + i, 0)
    out = pl.pallas_call(
        _graphcad_kernel,
        grid=(2, nblk),
        in_specs=[
            pl.BlockSpec((_BM, _N), rev),
            pl.BlockSpec((_N, _D), const),
            pl.BlockSpec((1, _D), const),
            pl.BlockSpec((1, _D), const),
            pl.BlockSpec((_D, _H), const),
            pl.BlockSpec((1, _H), const),
            pl.BlockSpec((1, 1), const),
            pl.BlockSpec((_H, _H), const),
            pl.BlockSpec((1, _H), const),
            pl.BlockSpec((1, 1), const),
            pl.BlockSpec((_H, _C), const),
            pl.BlockSpec((1, _C), const),
        ],
        out_specs=pl.BlockSpec((_BM, _C), rev),
        out_shape=jax.ShapeDtypeStruct((_N, _C), jnp.float32),
        scratch_shapes=[
            pltpu.VMEM((_N, _D), jnp.float32),
            pltpu.VMEM((_N, _D), jnp.float32),
        ],
    )(adj, feature, gamma2, beta2, W1, b1r, a1r, W2, b2r, a2r, W3, b3r)

    return out
